# baseline (speedup 1.0000x reference)
import functools

import jax
import jax.numpy as jnp
from jax.experimental import pallas as pl

_TILE_N = 512


def _kmeans_tile(x_ref, ct_ref, dist_ref, assign_ref, inertia_ref):
    x = x_ref[...]
    ct = ct_ref[...]
    cross = jax.lax.dot_general(
        x, ct, (((1,), (0,)), ((), ())),
        preferred_element_type=jnp.float32)
    x_sq = jnp.sum(x * x, axis=1, keepdims=True)
    c_sq = jnp.sum(ct * ct, axis=0, keepdims=True)
    d2 = jnp.maximum(x_sq + c_sq - 2.0 * cross, 0.0)
    dist = jnp.sqrt(d2)
    dist_ref[...] = dist
    assign_ref[...] = jnp.argmin(dist, axis=1).astype(jnp.int32)[:, None]
    mn = jnp.min(dist, axis=1)
    inertia_ref[...] = (mn * mn)[:, None]


@functools.partial(jax.jit, static_argnames=())
def kernel(data, centroids):
    n, f = data.shape
    k = centroids.shape[0]
    grid = (n // _TILE_N,)
    dist, assign, inertia = pl.pallas_call(
        _kmeans_tile,
        grid=grid,
        in_specs=[
            pl.BlockSpec((_TILE_N, f), lambda i: (i, 0)),
            pl.BlockSpec((f, k), lambda i: (0, 0)),
        ],
        out_specs=[
            pl.BlockSpec((_TILE_N, k), lambda i: (i, 0)),
            pl.BlockSpec((_TILE_N, 1), lambda i: (i, 0)),
            pl.BlockSpec((_TILE_N, 1), lambda i: (i, 0)),
        ],
        out_shape=[
            jax.ShapeDtypeStruct((n, k), jnp.float32),
            jax.ShapeDtypeStruct((n, 1), jnp.int32),
            jax.ShapeDtypeStruct((n, 1), jnp.float32),
        ],
    )(data, centroids.T)
    return dist, assign[:, 0], inertia[:, 0]
